# baseline (speedup 1.0000x reference)
import functools

import jax
import jax.numpy as jnp
from jax.experimental import pallas as pl
from jax.experimental.pallas import tpu as pltpu

N = 10000
E = 160000
IN_DIM = 386
HID = 512
OUT_DIM = 384
EDGE_DIM = 32
MAX_SP = 4
LAYERS = 4
HEADS = 8
C = HID // HEADS

_ROW_BLK = 1000


def _mm_kernel(x_ref, w_ref, b_ref, o_ref):
    o_ref[...] = (
        jnp.dot(x_ref[...], w_ref[...], preferred_element_type=jnp.float32)
        + b_ref[...]
    )


def _mm(x, w, b):
    n, k = x.shape
    m = w.shape[1]
    grid = (n // _ROW_BLK,)
    return pl.pallas_call(
        _mm_kernel,
        grid=grid,
        in_specs=[
            pl.BlockSpec((_ROW_BLK, k), lambda i: (i, 0)),
            pl.BlockSpec((k, m), lambda i: (0, 0)),
            pl.BlockSpec((1, m), lambda i: (0, 0)),
        ],
        out_specs=pl.BlockSpec((_ROW_BLK, m), lambda i: (i, 0)),
        out_shape=jax.ShapeDtypeStruct((n, m), jnp.float32),
    )(x, w, b.reshape(1, m))


def _conv_opt(h, ep_table, src, dst, eid, p):
    n = h.shape[0]
    q = _mm(h, p["Wq"], p["bq"]).reshape(n, HEADS, C)
    k = _mm(h, p["Wk"], p["bk"]).reshape(n, HEADS, C)
    v = _mm(h, p["Wv"], p["bv"]).reshape(n, HEADS, C)
    x_r = _mm(h, p["Wskip"], p["bskip"])
    ept = ep_table.reshape(MAX_SP, HEADS, C)

    qe = jnp.einsum("nhc,thc->nth", q, ept)
    dots = (q[dst] * k[src]).sum(-1)
    alpha = (dots + qe[dst, eid]) * (1.0 / jnp.sqrt(jnp.float32(C)))
    ex = jnp.exp(alpha)

    vj = v[src] + ept[eid]
    num = jax.ops.segment_sum(vj * ex[..., None], dst, num_segments=n)
    den = jax.ops.segment_sum(ex, dst, num_segments=n)
    out = (num / (den[..., None] + 1e-16)).reshape(n, HID)

    wb = p["Wbeta"].reshape(3, HID)
    wb_out = wb[0] + wb[2]
    wb_xr = wb[1] - wb[2]
    beta = jax.nn.sigmoid(out @ wb_out + x_r @ wb_xr)[:, None]
    return beta * x_r + (1.0 - beta) * out


def _ln(z, g, b):
    mu = z.mean(-1, keepdims=True)
    var = z.var(-1, keepdims=True)
    return (z - mu) / jnp.sqrt(var + 1e-5) * g + b


def kernel(x, params, edge_index, edge_attr_ids):
    src, dst = edge_index[0], edge_index[1]
    eid = edge_attr_ids
    xp = jnp.concatenate(
        [x, jnp.zeros((x.shape[0], 640 - IN_DIM), jnp.float32)], axis=1
    )
    in_wp = jnp.concatenate(
        [params["in_W"], jnp.zeros((640 - IN_DIM, HID), jnp.float32)], axis=0
    )
    h = _mm(xp, in_wp, params["in_b"])
    for p in params["layers"]:
        ep_table = params["edge_table"] @ p["We"]
        h2 = _conv_opt(h, ep_table, src, dst, eid, p)
        h = _ln(h + h2, p["ln_g"], p["ln_b"])
    return _mm(h, params["out_W"], params["out_b"])


# speedup vs baseline: 12.7567x; 12.7567x over previous
import functools

import jax
import jax.numpy as jnp
from jax import lax
from jax.experimental import pallas as pl
from jax.experimental.pallas import tpu as pltpu
from jax.experimental.pallas import tpu_sc as plsc

N = 10000
E = 160000
IN_DIM = 386
HID = 512
OUT_DIM = 384
EDGE_DIM = 32
MAX_SP = 4
LAYERS = 4
HEADS = 8
C = HID // HEADS

NC = 2
NS = 16
NW = NC * NS
EPW = E // NW
G = 40
NCHUNK = EPW // G
QW = 640
NPAD = 10112
RPT = NPAD // NS
SROWS = 2560
SPT = SROWS // NS
NCH2 = NCHUNK + 2
EPAD = E + 2 * G

_ROW_BLK = 1000


def _mm_kernel(x_ref, w_ref, b_ref, o_ref):
    o_ref[...] = (
        jnp.dot(x_ref[...], w_ref[...], preferred_element_type=jnp.float32)
        + b_ref[...]
    )


def _mm(x, w, b):
    n, k = x.shape
    m = w.shape[1]
    return pl.pallas_call(
        _mm_kernel,
        grid=(n // _ROW_BLK,),
        in_specs=[
            pl.BlockSpec((_ROW_BLK, k), lambda i: (i, 0)),
            pl.BlockSpec((k, m), lambda i: (0, 0)),
            pl.BlockSpec((1, m), lambda i: (0, 0)),
        ],
        out_specs=pl.BlockSpec((_ROW_BLK, m), lambda i: (i, 0)),
        out_shape=jax.ShapeDtypeStruct((n, m), jnp.float32),
    )(x, w, b.reshape(1, m))


_MESH = plsc.VectorSubcoreMesh(
    core_axis_name="c", subcore_axis_name="s", num_cores=NC, num_subcores=NS
)


@functools.partial(
    pl.kernel,
    out_type=jax.ShapeDtypeStruct((EPAD * 16,), jnp.float32),
    mesh=_MESH,
    compiler_params=pltpu.CompilerParams(needs_layout_passes=False),
    scratch_types=[
        pltpu.VMEM((NCHUNK, G), jnp.int32),
        pltpu.VMEM((NCHUNK, G), jnp.int32),
        pltpu.VMEM((EPW + 16,), jnp.int32),
        pltpu.VMEM((G, HID), jnp.float32),
        pltpu.VMEM((G, QW), jnp.float32),
        pltpu.VMEM((16 * G,), jnp.float32),
        pltpu.SemaphoreType.DMA,
        pltpu.SemaphoreType.DMA,
    ],
)
def _phase_a(qaug_hbm, k_hbm, src3_hbm, dst3_hbm, eid_hbm, ex_hbm,
             srcw, dstw, eidb, kb, qb, exb, sem1, sem2):
    wid = lax.axis_index("s") * NC + lax.axis_index("c")
    pltpu.sync_copy(src3_hbm.at[wid], srcw)
    pltpu.sync_copy(dst3_hbm.at[wid], dstw)
    pltpu.sync_copy(eid_hbm.at[pl.ds(wid * EPW, EPW)], eidb.at[pl.ds(0, EPW)])
    iota = lax.iota(jnp.int32, 16)

    def chunk(g, carry):
        cp1 = pltpu.async_copy(k_hbm.at[srcw.at[g]], kb, sem1)
        cp2 = pltpu.async_copy(qaug_hbm.at[dstw.at[g]], qb, sem2)
        cp1.wait()
        cp2.wait()

        def edge(e, carry2):
            srow = jnp.zeros((16,), jnp.float32)
            for h in range(HEADS):
                t = kb[e, pl.ds(h * 64, 16)] * qb[e, pl.ds(h * 64, 16)]
                for j in range(1, 4):
                    t = t + (kb[e, pl.ds(h * 64 + j * 16, 16)]
                             * qb[e, pl.ds(h * 64 + j * 16, 16)])
                s = plsc.cumsum(t)[15]
                srow = jnp.where(iota == h, s, srow)
            tid = eidb[pl.ds(g * G + e, 16)][0]
            qe = qb[e, pl.ds(HID + tid * 8, 16)]
            alpha = (srow + qe) * jnp.float32(0.125)
            exb[pl.ds(e * 16, 16)] = jnp.exp(alpha)
            return carry2

        lax.fori_loop(0, G, edge, 0)
        pltpu.sync_copy(
            exb, ex_hbm.at[pl.ds((wid * EPW + g * G) * 16, 16 * G)]
        )
        return carry

    lax.fori_loop(0, NCHUNK, chunk, 0)


@functools.partial(
    pl.kernel,
    out_type=(
        jax.ShapeDtypeStruct((NC, 4, NPAD, 128), jnp.float32),
        jax.ShapeDtypeStruct((NC, SROWS, 128), jnp.float32),
    ),
    mesh=_MESH,
    compiler_params=pltpu.CompilerParams(needs_layout_passes=False),
    scratch_types=[
        pltpu.VMEM_SHARED((NPAD, 128), jnp.float32),
        pltpu.VMEM((NCH2, G), jnp.int32),
        pltpu.VMEM((G + 1, 128), jnp.float32),
        pltpu.VMEM((G + 1, 128), jnp.float32),
        pltpu.VMEM((G + 1, 128), jnp.float32),
        pltpu.VMEM((16 * G,), jnp.float32),
        pltpu.VMEM((16 * G,), jnp.float32),
        pltpu.VMEM((16 * G,), jnp.float32),
        pltpu.VMEM((G,), jnp.int32),
        pltpu.VMEM((G,), jnp.int32),
        pltpu.VMEM((G,), jnp.int32),
        pltpu.VMEM((G + 16,), jnp.int32),
        pltpu.VMEM((G + 16,), jnp.int32),
        pltpu.VMEM((G + 16,), jnp.int32),
        pltpu.SemaphoreType.DMA,
        pltpu.SemaphoreType.DMA,
        pltpu.SemaphoreType.DMA,
        pltpu.SemaphoreType.DMA,
        pltpu.SemaphoreType.DMA,
        pltpu.SemaphoreType.DMA,
    ],
)
def _phase_b(vt_hbm, vidx4_hbm, dst1_hbm, dstden2_hbm, dmod2_hbm,
             ex_hbm, zeros_hbm, acc_hbm, s_hbm,
             accs, vidxw, vb0, vb1, vb2, exb0, exb1, exb2,
             dstb0, dstb1, dstb2, dmb0, dmb1, dmb2,
             sgl0, sgl1, sgl2, ss0, ss1, ss2):
    cid = lax.axis_index("c")
    sid = lax.axis_index("s")
    wid = sid * NC + cid
    iota = lax.iota(jnp.int32, 16)
    zero16 = jnp.zeros((16,), jnp.float32)
    zero16i = jnp.zeros((16,), jnp.int32)
    vbs = (vb0, vb1, vb2)
    exbs = (exb0, exb1, exb2)
    dstbs = (dstb0, dstb1, dstb2)
    dmbs = (dmb0, dmb1, dmb2)
    sgls = (sgl0, sgl1, sgl2)
    sss = (ss0, ss1, ss2)

    def zero_vb(par):
        def zr(e, c):
            for j in range(8):
                vbs[par][e, pl.ds(j * 16, 16)] = zero16
            return c
        lax.fori_loop(0, G + 1, zr, 0)

    def zero_dstb(par):
        dstbs[par][pl.ds(0, 16)] = zero16i
        dstbs[par][pl.ds(16, 16)] = zero16i
        dstbs[par][pl.ds(24, 16)] = zero16i

    def issue_v_loads(g, par):
        base = wid * EPW + g * G
        pltpu.async_copy(
            ex_hbm.at[pl.ds(base * 16, 16 * G)], exbs[par], sgls[par])
        pltpu.async_copy(
            dst1_hbm.at[pl.ds(base, G)], dstbs[par], sgls[par])
        pltpu.async_copy(
            vt_hbm.at[vidxw.at[g]], vbs[par].at[pl.ds(0, G)], sgls[par])

    def drain_v_loads(g, par):
        base = wid * EPW + g * G
        pltpu.make_async_copy(
            ex_hbm.at[pl.ds(base * 16, 16 * G)], exbs[par], sgls[par]).wait()
        pltpu.make_async_copy(
            dst1_hbm.at[pl.ds(base, G)], dstbs[par], sgls[par]).wait()
        pltpu.make_async_copy(
            vt_hbm.at[pl.ds(0, G)], vbs[par].at[pl.ds(0, G)],
            sgls[par]).wait()

    def drain_scatter(par):
        pltpu.make_async_copy(
            vt_hbm.at[pl.ds(0, G)], vbs[par].at[pl.ds(0, G)], sss[par]).wait()

    for cb in range(4):
        pltpu.sync_copy(vidx4_hbm.at[cb, wid], vidxw)
        pltpu.sync_copy(zeros_hbm, accs.at[pl.ds(sid * RPT, RPT)])
        plsc.subcore_barrier()

        zero_vb(2)
        zero_dstb(2)
        pltpu.async_copy(
            vbs[2].at[pl.ds(0, G)], accs.at[dstbs[2]], sss[2], add=True)
        issue_v_loads(0, 0)
        issue_v_loads(1, 1)

        def vbody(g, par):
            drain_v_loads(g, par)
            exbp = exbs[par]
            vbp = vbs[par]

            def edge(e, carry2):
                exv = exbp[pl.ds(e * 16, 16)]
                ex0 = exv[2 * cb]
                ex1 = exv[2 * cb + 1]
                for j in range(8):
                    exh = ex0 if j < 4 else ex1
                    vbp[e, pl.ds(j * 16, 16)] = exh * vbp[e, pl.ds(j * 16, 16)]
                return carry2

            lax.fori_loop(0, G, edge, 0)
            pltpu.async_copy(
                vbp.at[pl.ds(0, G)], accs.at[dstbs[par]], sss[par], add=True)
            nxt = (par + 2) % 3
            drain_scatter(nxt)
            issue_v_loads(g + 2, nxt)

        def triple(i, carry):
            vbody(3 * i, 0)
            vbody(3 * i + 1, 1)
            vbody(3 * i + 2, 2)
            return carry

        lax.fori_loop(0, NCHUNK // 3, triple, 0)
        vbody(NCHUNK - 2, 0)
        vbody(NCHUNK - 1, 1)
        drain_scatter(1)
        drain_v_loads(NCHUNK, 2)
        drain_v_loads(NCHUNK + 1, 0)
        plsc.subcore_barrier()
        pltpu.sync_copy(
            accs.at[pl.ds(sid * RPT, RPT)],
            acc_hbm.at[cid, cb, pl.ds(sid * RPT, RPT)],
        )
        plsc.subcore_barrier()

    pltpu.sync_copy(dstden2_hbm.at[wid], vidxw)
    pltpu.sync_copy(
        zeros_hbm.at[pl.ds(0, SPT)], accs.at[pl.ds(sid * SPT, SPT)]
    )
    plsc.subcore_barrier()

    def issue_s_loads(g, par):
        base = wid * EPW + g * G
        pltpu.async_copy(
            ex_hbm.at[pl.ds(base * 16, 16 * G)], exbs[par], sgls[par])
        pltpu.async_copy(
            dmod2_hbm.at[pl.ds(base, G)], dmbs[par].at[pl.ds(0, G)],
            sgls[par])

    def drain_s_loads(g, par):
        base = wid * EPW + g * G
        pltpu.make_async_copy(
            ex_hbm.at[pl.ds(base * 16, 16 * G)], exbs[par], sgls[par]).wait()
        pltpu.make_async_copy(
            dmod2_hbm.at[pl.ds(base, G)], dmbs[par].at[pl.ds(0, G)],
            sgls[par]).wait()

    zero_vb(2)
    zero_dstb(2)
    pltpu.async_copy(
        vbs[2].at[pl.ds(0, G)], accs.at[dstbs[2]], sss[2], add=True)
    issue_s_loads(0, 0)
    issue_s_loads(1, 1)

    def sbody(g, par):
        drain_s_loads(g, par)
        exbp = exbs[par]
        dmbp = dmbs[par]
        vbp = vbs[par]

        def edge(e, carry2):
            exv = exbp[pl.ds(e * 16, 16)]
            dm = dmbp[pl.ds(e, 16)][0]
            exm = jnp.where(iota < 8, exv, 0.0)
            for j in range(8):
                vbp[e, pl.ds(j * 16, 16)] = zero16
            vbp[e, pl.ds(dm, 16)] = exm
            return carry2

        lax.fori_loop(0, G, edge, 0)
        pltpu.async_copy(
            vbp.at[pl.ds(0, G)], accs.at[vidxw.at[g]], sss[par], add=True)
        nxt = (par + 2) % 3
        drain_scatter(nxt)
        issue_s_loads(g + 2, nxt)

    def striple(i, carry):
        sbody(3 * i, 0)
        sbody(3 * i + 1, 1)
        sbody(3 * i + 2, 2)
        return carry

    lax.fori_loop(0, NCHUNK // 3, striple, 0)
    sbody(NCHUNK - 2, 0)
    sbody(NCHUNK - 1, 1)
    drain_scatter(1)
    drain_s_loads(NCHUNK, 2)
    drain_s_loads(NCHUNK + 1, 0)
    plsc.subcore_barrier()
    pltpu.sync_copy(
        accs.at[pl.ds(sid * SPT, SPT)],
        s_hbm.at[cid, pl.ds(sid * SPT, SPT)],
    )


def _conv(h, ep_table, src3, dst3, eid1, vidx4, dst1p, dstden2, dmod2,
          zeros_acc, p):
    n = h.shape[0]
    q = _mm(h, p["Wq"], p["bq"])
    k = _mm(h, p["Wk"], p["bk"])
    v = _mm(h, p["Wv"], p["bv"])
    x_r = _mm(h, p["Wskip"], p["bskip"])

    hh = jnp.arange(HID) // C
    wqe = jnp.zeros((HID, MAX_SP * HEADS), jnp.float32)
    cols = jnp.arange(MAX_SP)[None, :] * HEADS + hh[:, None]
    wqe = wqe.at[jnp.arange(HID)[:, None], cols].set(ep_table.T)
    qe = q @ wqe
    qaug = jnp.concatenate(
        [q, qe, jnp.zeros((n, QW - HID - MAX_SP * HEADS), jnp.float32)], axis=1
    )

    ex = _phase_a(qaug, k, src3, dst3, eid1)

    vt = v.reshape(n, 4, 128).transpose(1, 0, 2).reshape(4 * n, 128)
    acc, s_packed = _phase_b(
        vt, vidx4, dst1p, dstden2, dmod2, ex, zeros_acc
    )

    summed = (acc[0] + acc[1])[:, :n, :]
    num = summed.transpose(1, 0, 2).reshape(n, HID)
    s_mat = (s_packed[0] + s_packed[1]).reshape(SROWS * 4, 32)[:n]
    den = s_mat.reshape(n, MAX_SP, HEADS).sum(1)
    headmask = (jnp.arange(HID) // C)[None, :] == jnp.arange(HEADS)[:, None]
    w2 = (ep_table[:, None, :] * headmask[None].astype(jnp.float32))
    w2 = w2.reshape(MAX_SP * HEADS, HID)
    num = num + s_mat @ w2
    den64 = jnp.repeat(den, C, axis=1)
    out = num / (den64 + 1e-16)

    wb = p["Wbeta"].reshape(3, HID)
    wb_out = wb[0] + wb[2]
    wb_xr = wb[1] - wb[2]
    beta = jax.nn.sigmoid(out @ wb_out + x_r @ wb_xr)[:, None]
    return beta * x_r + (1.0 - beta) * out


def _ln(z, g, b):
    mu = z.mean(-1, keepdims=True)
    var = z.var(-1, keepdims=True)
    return (z - mu) / jnp.sqrt(var + 1e-5) * g + b


def kernel(x, params, edge_index, edge_attr_ids):
    src, dst = edge_index[0], edge_index[1]
    eid = edge_attr_ids.astype(jnp.int32)
    src3 = src.reshape(NW, NCHUNK, G)
    dst3 = dst.reshape(NW, NCHUNK, G)
    vidx4 = (jnp.arange(4, dtype=jnp.int32)[:, None] * N + src[None, :])
    vidx4 = vidx4.reshape(4, NW, NCHUNK, G)
    vidx4 = jnp.pad(vidx4, ((0, 0), (0, 0), (0, 2), (0, 0)))
    dst1p = jnp.pad(dst, (0, 2 * G))
    dstden2 = jnp.pad(
        (dst // 4).reshape(NW, NCHUNK, G), ((0, 0), (0, 2), (0, 0))
    )
    dmod2 = jnp.pad(((dst % 4) * 32 + eid * 8).astype(jnp.int32), (0, 2 * G))
    zeros_acc = jnp.zeros((RPT, 128), jnp.float32)

    xp = jnp.concatenate(
        [x, jnp.zeros((x.shape[0], 640 - IN_DIM), jnp.float32)], axis=1
    )
    in_wp = jnp.concatenate(
        [params["in_W"], jnp.zeros((640 - IN_DIM, HID), jnp.float32)], axis=0
    )
    h = _mm(xp, in_wp, params["in_b"])
    for p in params["layers"]:
        ep_table = params["edge_table"] @ p["We"]
        h2 = _conv(h, ep_table, src3, dst3, eid, vidx4, dst1p, dstden2,
                   dmod2, zeros_acc, p)
        h = _ln(h + h2, p["ln_g"], p["ln_b"])
    return _mm(h, params["out_W"], params["out_b"])
